# baseline (speedup 1.0000x reference)
import functools

import jax
import jax.numpy as jnp
from jax import lax
from jax.experimental import pallas as pl
from jax.experimental.pallas import tpu as pltpu
from jax.experimental.pallas import tpu_sc as plsc

D = 64
B = 16384
MARGIN = 1.0

NC = 2
NS = 16
NW = NC * NS

NIDX = 4 * B
E_PER_W = NIDX // NW
R_PER_W = B // NW
CHUNK = 512
E_CHUNKS = E_PER_W // CHUNK

BLK = 2048
GRID = B // BLK


def _gather_rows(ent, rel, eidx, ridx):
    mesh = plsc.VectorSubcoreMesh(core_axis_name="core", subcore_axis_name="subcore")

    @functools.partial(
        pl.kernel,
        out_type=(
            jax.ShapeDtypeStruct((NIDX, D), jnp.float32),
            jax.ShapeDtypeStruct((B, D), jnp.float32),
        ),
        mesh=mesh,
        scratch_types=[
            pltpu.VMEM((CHUNK,), jnp.int32),
            pltpu.SemaphoreType.DMA,
        ],
    )
    def gk(ent_hbm, rel_hbm, eidx_hbm, ridx_hbm, ent_out, rel_out, idx_v, sem):
        wid = lax.axis_index("subcore") * NC + lax.axis_index("core")

        def chunk_gather(tab_hbm, idx_hbm, out_hbm, base):
            pltpu.sync_copy(idx_hbm.at[pl.ds(base, CHUNK)], idx_v)

            @pl.loop(0, CHUNK // 16)
            def _(g):
                vec = idx_v[pl.ds(g * 16, 16)]
                for k in range(16):
                    i = vec[k]
                    pltpu.async_copy(
                        tab_hbm.at[pl.ds(i, 1)],
                        out_hbm.at[pl.ds(base + g * 16 + k, 1)], sem)

            @pl.loop(0, CHUNK)
            def _(j):
                pltpu.make_async_copy(
                    tab_hbm.at[pl.ds(0, 1)],
                    out_hbm.at[pl.ds(base + j, 1)], sem).wait()

        @pl.loop(0, E_CHUNKS)
        def _(c):
            chunk_gather(ent_hbm, eidx_hbm, ent_out, wid * E_PER_W + c * CHUNK)

        chunk_gather(rel_hbm, ridx_hbm, rel_out, wid * R_PER_W)

    return gk(ent, rel, eidx, ridx)


def _loss_body(h_ref, t_ref, hn_ref, tn_ref, r_ref, out_ref):
    i = pl.program_id(0)

    def nrm(x):
        n = jnp.sqrt(jnp.sum(x * x, axis=1, keepdims=True))
        return x / (n + 1e-12)

    h = nrm(h_ref[...])
    t = nrm(t_ref[...])
    hn = nrm(hn_ref[...])
    tn = nrm(tn_ref[...])
    r = r_ref[...]
    pos = jnp.sum(jnp.abs(h + r - t), axis=1)
    neg = jnp.sum(jnp.abs(hn + r - tn), axis=1)
    part = jnp.sum(jnp.maximum(MARGIN + pos - neg, 0.0)) * (1.0 / B)

    @pl.when(i == 0)
    def _():
        out_ref[...] = jnp.zeros_like(out_ref)

    out_ref[...] += jnp.reshape(part, (1, 1))


def kernel(entity_embedding, relation_embedding, pos_h, pos_r, pos_t, neg_h, neg_t):
    eidx = jnp.concatenate([pos_h, pos_t, neg_h, neg_t])
    g_ent, g_rel = _gather_rows(entity_embedding, relation_embedding, eidx, pos_r)

    loss = pl.pallas_call(
        _loss_body,
        grid=(GRID,),
        in_specs=[
            pl.BlockSpec((BLK, D), lambda i: (i, 0)),
            pl.BlockSpec((BLK, D), lambda i: (i + GRID, 0)),
            pl.BlockSpec((BLK, D), lambda i: (i + 2 * GRID, 0)),
            pl.BlockSpec((BLK, D), lambda i: (i + 3 * GRID, 0)),
            pl.BlockSpec((BLK, D), lambda i: (i, 0)),
        ],
        out_specs=pl.BlockSpec((1, 1), lambda i: (0, 0)),
        out_shape=jax.ShapeDtypeStruct((1, 1), jnp.float32),
    )(g_ent, g_ent, g_ent, g_ent, g_rel)
    return loss[0, 0]


# speedup vs baseline: 1.8710x; 1.8710x over previous
import dataclasses
import functools

import jax
import jax.numpy as jnp
from jax import lax
from jax.experimental import pallas as pl
from jax.experimental.pallas import tpu as pltpu
from jax.experimental.pallas import tpu_sc as plsc

D = 64
B = 16384
MARGIN = 1.0

NC = 2
NS = 16
NW = NC * NS

NIDX = 4 * B
E_PER_W = NIDX // NW
R_PER_W = B // NW
CH = 128
E_CHUNKS = E_PER_W // CH
R_CHUNKS = R_PER_W // CH

BLK = 2048
GRID = B // BLK


def _sc_compiler_params():
    cp = pltpu.CompilerParams()
    if "needs_layout_passes" in pltpu.CompilerParams.__dataclass_fields__:
        cp = dataclasses.replace(cp, needs_layout_passes=False)
    return cp


def _gather_rows(ent2, rel2, eidx2d, ridx2d):
    mesh = plsc.VectorSubcoreMesh(core_axis_name="core", subcore_axis_name="subcore")

    @functools.partial(
        pl.kernel,
        out_type=(
            jax.ShapeDtypeStruct((NIDX, D), jnp.float32),
            jax.ShapeDtypeStruct((B, D), jnp.float32),
        ),
        mesh=mesh,
        scratch_types=[
            pltpu.VMEM((CH,), jnp.int32),
            pltpu.VMEM((CH,), jnp.int32),
            pltpu.VMEM((CH, 2 * D), jnp.float32),
            pltpu.VMEM((CH, D), jnp.float32),
            pltpu.SemaphoreType.DMA,
        ],
        compiler_params=_sc_compiler_params(),
    )
    def gk(ent_hbm, rel_hbm, eidx_hbm, ridx_hbm, ent_out, rel_out,
           idx_v, hi_v, rows_v, out_v, sem):
        wid = lax.axis_index("subcore") * NC + lax.axis_index("core")

        def chunk_gather(tab_hbm, idx_hbm, out_hbm, row, base):
            pltpu.sync_copy(idx_hbm.at[row], idx_v)

            @pl.loop(0, CH // 16)
            def _(g):
                v = idx_v[pl.ds(g * 16, 16)]
                hi_v[pl.ds(g * 16, 16)] = lax.shift_right_logical(v, 1)

            pltpu.async_copy(tab_hbm.at[hi_v], rows_v, sem).wait()

            @pl.loop(0, CH // 16)
            def _(g):
                rid = g * 16 + lax.iota(jnp.int32, 16)
                par = (idx_v[pl.ds(g * 16, 16)] & 1) * D
                for c in range(D):
                    vals = plsc.load_gather(rows_v, [rid, par + c])
                    plsc.store_scatter(
                        out_v, [rid, lax.full((16,), c, jnp.int32)], vals)

            pltpu.sync_copy(out_v, out_hbm.at[pl.ds(base, CH)])

        @pl.loop(0, E_CHUNKS)
        def _(c):
            row = wid * E_CHUNKS + c
            chunk_gather(ent_hbm, eidx_hbm, ent_out, row, row * CH)

        @pl.loop(0, R_CHUNKS)
        def _(c):
            row = wid * R_CHUNKS + c
            chunk_gather(rel_hbm, ridx_hbm, rel_out, row, row * CH)

    return gk(ent2, rel2, eidx2d, ridx2d)


def _loss_body(h_ref, t_ref, hn_ref, tn_ref, r_ref, out_ref):
    i = pl.program_id(0)

    def nrm(x):
        n = jnp.sqrt(jnp.sum(x * x, axis=1, keepdims=True))
        return x / (n + 1e-12)

    h = nrm(h_ref[...])
    t = nrm(t_ref[...])
    hn = nrm(hn_ref[...])
    tn = nrm(tn_ref[...])
    r = r_ref[...]
    pos = jnp.sum(jnp.abs(h + r - t), axis=1)
    neg = jnp.sum(jnp.abs(hn + r - tn), axis=1)
    part = jnp.sum(jnp.maximum(MARGIN + pos - neg, 0.0)) * (1.0 / B)

    @pl.when(i == 0)
    def _():
        out_ref[...] = jnp.zeros_like(out_ref)

    out_ref[...] += jnp.reshape(part, (1, 1))


def kernel(entity_embedding, relation_embedding, pos_h, pos_r, pos_t, neg_h, neg_t):
    ent2 = jnp.reshape(entity_embedding, (entity_embedding.shape[0] // 2, 2 * D))
    rel2 = jnp.reshape(relation_embedding, (relation_embedding.shape[0] // 2, 2 * D))
    eidx2d = jnp.concatenate([pos_h, pos_t, neg_h, neg_t]).reshape(NIDX // CH, CH)
    ridx2d = pos_r.reshape(B // CH, CH)
    g_ent, g_rel = _gather_rows(ent2, rel2, eidx2d, ridx2d)

    loss = pl.pallas_call(
        _loss_body,
        grid=(GRID,),
        in_specs=[
            pl.BlockSpec((BLK, D), lambda i: (i, 0)),
            pl.BlockSpec((BLK, D), lambda i: (i + GRID, 0)),
            pl.BlockSpec((BLK, D), lambda i: (i + 2 * GRID, 0)),
            pl.BlockSpec((BLK, D), lambda i: (i + 3 * GRID, 0)),
            pl.BlockSpec((BLK, D), lambda i: (i, 0)),
        ],
        out_specs=pl.BlockSpec((1, 1), lambda i: (0, 0)),
        out_shape=jax.ShapeDtypeStruct((1, 1), jnp.float32),
    )(g_ent, g_ent, g_ent, g_ent, g_rel)
    return loss[0, 0]


# speedup vs baseline: 2.2752x; 1.2160x over previous
import dataclasses
import functools

import jax
import jax.numpy as jnp
from jax import lax
from jax.experimental import pallas as pl
from jax.experimental.pallas import tpu as pltpu
from jax.experimental.pallas import tpu_sc as plsc

N_ENT = 1000000
N_REL = 1000
HALF_ENT = N_ENT // 2
HALF_REL = N_REL // 2
D = 64
B = 16384
MARGIN = 1.0

NC = 2
NS = 16
NW = NC * NS

NIDX = 4 * B
E_PER_W = NIDX // NW
R_PER_W = B // NW
CH = 128
E_CHUNKS = E_PER_W // CH
R_CHUNKS = R_PER_W // CH

PACK_BLK = 5000
PACK_GRID = HALF_ENT // PACK_BLK

BLK = 2048
GRID = B // BLK


def _sc_compiler_params():
    cp = pltpu.CompilerParams()
    if "needs_layout_passes" in pltpu.CompilerParams.__dataclass_fields__:
        cp = dataclasses.replace(cp, needs_layout_passes=False)
    return cp


def _pack_body(a_ref, b_ref, o_ref):
    o_ref[:, :D] = a_ref[...]
    o_ref[:, D:] = b_ref[...]


def _pack_entity(ent):
    return pl.pallas_call(
        _pack_body,
        grid=(PACK_GRID,),
        in_specs=[
            pl.BlockSpec((PACK_BLK, D), lambda i: (i, 0)),
            pl.BlockSpec((PACK_BLK, D), lambda i: (i + PACK_GRID, 0)),
        ],
        out_specs=pl.BlockSpec((PACK_BLK, 2 * D), lambda i: (i, 0)),
        out_shape=jax.ShapeDtypeStruct((HALF_ENT, 2 * D), jnp.float32),
    )(ent, ent)


def _gather_rows(ent2, rel2, eidx2d, ridx2d):
    mesh = plsc.VectorSubcoreMesh(core_axis_name="core", subcore_axis_name="subcore")

    @functools.partial(
        pl.kernel,
        out_type=(
            jax.ShapeDtypeStruct((NIDX, 2 * D), jnp.float32),
            jax.ShapeDtypeStruct((B, 2 * D), jnp.float32),
        ),
        mesh=mesh,
        scratch_types=[
            pltpu.VMEM((CH,), jnp.int32),
            pltpu.VMEM((CH, 2 * D), jnp.float32),
            pltpu.SemaphoreType.DMA,
        ],
        compiler_params=_sc_compiler_params(),
    )
    def gk(ent_hbm, rel_hbm, eidx_hbm, ridx_hbm, ent_out, rel_out,
           idx_v, rows_v, sem):
        wid = lax.axis_index("subcore") * NC + lax.axis_index("core")

        def chunk_gather(tab_hbm, idx_hbm, out_hbm, row):
            pltpu.sync_copy(idx_hbm.at[row], idx_v)
            pltpu.async_copy(tab_hbm.at[idx_v], rows_v, sem).wait()
            pltpu.sync_copy(rows_v, out_hbm.at[pl.ds(row * CH, CH)])

        @pl.loop(0, E_CHUNKS)
        def _(c):
            chunk_gather(ent_hbm, eidx_hbm, ent_out, wid * E_CHUNKS + c)

        @pl.loop(0, R_CHUNKS)
        def _(c):
            chunk_gather(rel_hbm, ridx_hbm, rel_out, wid * R_CHUNKS + c)

    return gk(ent2, rel2, eidx2d, ridx2d)


def _loss_body(h_ref, t_ref, hn_ref, tn_ref, r_ref,
               ph_ref, pt_ref, phn_ref, ptn_ref, pr_ref, out_ref):
    i = pl.program_id(0)

    def pick(x2_ref, p_ref):
        x2 = x2_ref[...]
        p = p_ref[...]
        return jnp.where(p != 0, x2[:, D:], x2[:, :D])

    def nrm(x):
        n = jnp.sqrt(jnp.sum(x * x, axis=1, keepdims=True))
        return x / (n + 1e-12)

    h = nrm(pick(h_ref, ph_ref))
    t = nrm(pick(t_ref, pt_ref))
    hn = nrm(pick(hn_ref, phn_ref))
    tn = nrm(pick(tn_ref, ptn_ref))
    r = pick(r_ref, pr_ref)
    pos = jnp.sum(jnp.abs(h + r - t), axis=1)
    neg = jnp.sum(jnp.abs(hn + r - tn), axis=1)
    part = jnp.sum(jnp.maximum(MARGIN + pos - neg, 0.0)) * (1.0 / B)

    @pl.when(i == 0)
    def _():
        out_ref[...] = jnp.zeros_like(out_ref)

    out_ref[...] += jnp.reshape(part, (1, 1))


def kernel(entity_embedding, relation_embedding, pos_h, pos_r, pos_t, neg_h, neg_t):
    ent2 = _pack_entity(entity_embedding)
    rel2 = jnp.reshape(relation_embedding, (HALF_REL, 2 * D))

    eidx = jnp.concatenate([pos_h, pos_t, neg_h, neg_t])
    epair = jnp.where(eidx >= HALF_ENT, eidx - HALF_ENT, eidx)
    epar = (eidx >= HALF_ENT).astype(jnp.int32)
    rpair = pos_r >> 1
    rpar = pos_r & 1

    g_ent, g_rel = _gather_rows(
        ent2, rel2,
        epair.reshape(NIDX // CH, CH),
        rpair.reshape(B // CH, CH),
    )

    pars2d = jnp.concatenate([epar, rpar]).reshape(5 * B, 1)

    loss = pl.pallas_call(
        _loss_body,
        grid=(GRID,),
        in_specs=[
            pl.BlockSpec((BLK, 2 * D), lambda i: (i, 0)),
            pl.BlockSpec((BLK, 2 * D), lambda i: (i + GRID, 0)),
            pl.BlockSpec((BLK, 2 * D), lambda i: (i + 2 * GRID, 0)),
            pl.BlockSpec((BLK, 2 * D), lambda i: (i + 3 * GRID, 0)),
            pl.BlockSpec((BLK, 2 * D), lambda i: (i, 0)),
            pl.BlockSpec((BLK, 1), lambda i: (i, 0)),
            pl.BlockSpec((BLK, 1), lambda i: (i + GRID, 0)),
            pl.BlockSpec((BLK, 1), lambda i: (i + 2 * GRID, 0)),
            pl.BlockSpec((BLK, 1), lambda i: (i + 3 * GRID, 0)),
            pl.BlockSpec((BLK, 1), lambda i: (i + 4 * GRID, 0)),
        ],
        out_specs=pl.BlockSpec((1, 1), lambda i: (0, 0)),
        out_shape=jax.ShapeDtypeStruct((1, 1), jnp.float32),
    )(g_ent, g_ent, g_ent, g_ent, g_rel,
      pars2d, pars2d, pars2d, pars2d, pars2d)
    return loss[0, 0]


# speedup vs baseline: 2.7525x; 1.2098x over previous
import dataclasses
import functools

import jax
import jax.numpy as jnp
from jax import lax
from jax.experimental import pallas as pl
from jax.experimental.pallas import tpu as pltpu
from jax.experimental.pallas import tpu_sc as plsc

N_ENT = 1000000
N_REL = 1000
HALF_ENT = N_ENT // 2
HALF_REL = N_REL // 2
D = 64
B = 16384
MARGIN = 1.0

NC = 2
NS = 16
NW = NC * NS

NIDX = 4 * B
E_PER_W = NIDX // NW
R_PER_W = B // NW
CH = 128
E_CHUNKS = E_PER_W // CH
R_CHUNKS = R_PER_W // CH

PACK_BLK = 5000
PACK_GRID = HALF_ENT // PACK_BLK

BLK = 2048
GRID = B // BLK


def _sc_compiler_params():
    cp = pltpu.CompilerParams()
    if "needs_layout_passes" in pltpu.CompilerParams.__dataclass_fields__:
        cp = dataclasses.replace(cp, needs_layout_passes=False)
    return cp


def _pack_body(x_ref, o_ref):
    o_ref[:, :D] = x_ref[0]
    o_ref[:, D:] = x_ref[1]


def _pack_entity(ent):
    ent3 = jnp.reshape(ent, (2, HALF_ENT, D))
    return pl.pallas_call(
        _pack_body,
        grid=(PACK_GRID,),
        in_specs=[pl.BlockSpec((2, PACK_BLK, D), lambda i: (0, i, 0))],
        out_specs=pl.BlockSpec((PACK_BLK, 2 * D), lambda i: (i, 0)),
        out_shape=jax.ShapeDtypeStruct((HALF_ENT, 2 * D), jnp.float32),
    )(ent3)


def _gather_rows(ent2, rel2, eidx2d, ridx2d):
    mesh = plsc.VectorSubcoreMesh(core_axis_name="core", subcore_axis_name="subcore")

    @functools.partial(
        pl.kernel,
        out_type=(
            jax.ShapeDtypeStruct((NIDX, 2 * D), jnp.float32),
            jax.ShapeDtypeStruct((B, 2 * D), jnp.float32),
        ),
        mesh=mesh,
        scratch_types=[
            pltpu.VMEM((CH,), jnp.int32),
            pltpu.VMEM((CH, 2 * D), jnp.float32),
            pltpu.SemaphoreType.DMA,
        ],
        compiler_params=_sc_compiler_params(),
    )
    def gk(ent_hbm, rel_hbm, eidx_hbm, ridx_hbm, ent_out, rel_out,
           idx_v, rows_v, sem):
        wid = lax.axis_index("subcore") * NC + lax.axis_index("core")

        def chunk_gather(tab_hbm, idx_hbm, out_hbm, row):
            pltpu.sync_copy(idx_hbm.at[row], idx_v)
            pltpu.async_copy(tab_hbm.at[idx_v], rows_v, sem).wait()
            pltpu.sync_copy(rows_v, out_hbm.at[pl.ds(row * CH, CH)])

        @pl.loop(0, E_CHUNKS)
        def _(c):
            chunk_gather(ent_hbm, eidx_hbm, ent_out, wid * E_CHUNKS + c)

        @pl.loop(0, R_CHUNKS)
        def _(c):
            chunk_gather(rel_hbm, ridx_hbm, rel_out, wid * R_CHUNKS + c)

    return gk(ent2, rel2, eidx2d, ridx2d)


def _loss_body(e_ref, r_ref, ph_ref, pt_ref, phn_ref, ptn_ref, pr_ref, out_ref):
    i = pl.program_id(0)

    def pick(x2, p_ref):
        p = p_ref[...]
        return jnp.where(p != 0, x2[:, D:], x2[:, :D])

    def nrm(x):
        n = jnp.sqrt(jnp.sum(x * x, axis=1, keepdims=True))
        return x / (n + 1e-12)

    h = nrm(pick(e_ref[0], ph_ref))
    t = nrm(pick(e_ref[1], pt_ref))
    hn = nrm(pick(e_ref[2], phn_ref))
    tn = nrm(pick(e_ref[3], ptn_ref))
    r = pick(r_ref[...], pr_ref)
    pos = jnp.sum(jnp.abs(h + r - t), axis=1)
    neg = jnp.sum(jnp.abs(hn + r - tn), axis=1)
    part = jnp.sum(jnp.maximum(MARGIN + pos - neg, 0.0)) * (1.0 / B)

    @pl.when(i == 0)
    def _():
        out_ref[...] = jnp.zeros_like(out_ref)

    out_ref[...] += jnp.reshape(part, (1, 1))


def kernel(entity_embedding, relation_embedding, pos_h, pos_r, pos_t, neg_h, neg_t):
    ent2 = _pack_entity(entity_embedding)
    rel2 = jnp.reshape(relation_embedding, (HALF_REL, 2 * D))

    eidx = jnp.concatenate([pos_h, pos_t, neg_h, neg_t])
    epair = jnp.where(eidx >= HALF_ENT, eidx - HALF_ENT, eidx)
    epar = (eidx >= HALF_ENT).astype(jnp.int32)
    rpair = pos_r >> 1
    rpar = pos_r & 1

    g_ent, g_rel = _gather_rows(
        ent2, rel2,
        epair.reshape(NIDX // CH, CH),
        rpair.reshape(B // CH, CH),
    )

    pars2d = jnp.concatenate([epar, rpar]).reshape(5 * B, 1)

    g_ent4 = jnp.reshape(g_ent, (4, B, 2 * D))
    loss = pl.pallas_call(
        _loss_body,
        grid=(GRID,),
        in_specs=[
            pl.BlockSpec((4, BLK, 2 * D), lambda i: (0, i, 0)),
            pl.BlockSpec((BLK, 2 * D), lambda i: (i, 0)),
            pl.BlockSpec((BLK, 1), lambda i: (i, 0)),
            pl.BlockSpec((BLK, 1), lambda i: (i + GRID, 0)),
            pl.BlockSpec((BLK, 1), lambda i: (i + 2 * GRID, 0)),
            pl.BlockSpec((BLK, 1), lambda i: (i + 3 * GRID, 0)),
            pl.BlockSpec((BLK, 1), lambda i: (i + 4 * GRID, 0)),
        ],
        out_specs=pl.BlockSpec((1, 1), lambda i: (0, 0)),
        out_shape=jax.ShapeDtypeStruct((1, 1), jnp.float32),
    )(g_ent4, g_rel, pars2d, pars2d, pars2d, pars2d, pars2d)
    return loss[0, 0]
